# baseline (speedup 1.0000x reference)
import functools

import jax
import jax.numpy as jnp
from jax.experimental import pallas as pl
from jax.experimental.pallas import tpu as pltpu


def _head_kernel(qk_ref, v_ref, anchors_ref, w_ref, out_ref, *, n, nbits):
    x = qk_ref[0, 0]
    v = v_ref[0, 0]
    a = anchors_ref[0, 0]
    w = w_ref[0]

    x = x / jnp.sqrt(jnp.sum(x * x, axis=-1, keepdims=True))

    sim = jax.lax.dot_general(x, a, (((1,), (1,)), ((), ())),
                              preferred_element_type=jnp.float32)
    d2 = jnp.clip(2.0 - 2.0 * sim, 0.0, None)
    kf = jnp.exp(-0.5 * d2)
    kc = kf - jnp.mean(kf, axis=0, keepdims=True)
    codes = jnp.tanh(
        jax.lax.dot_general(kc, w, (((1,), (0,)), ((), ())),
                            preferred_element_type=jnp.float32))

    bias = float(nbits + 1)
    k_cumsum = jnp.sum(codes, axis=0, keepdims=True)
    denom = jnp.sum(codes * k_cumsum, axis=1, keepdims=True)
    d_inv = 1.0 / (denom + n * bias)
    context = jax.lax.dot_general(codes, v, (((0,), (0,)), ((), ())),
                                  preferred_element_type=jnp.float32)
    out = jax.lax.dot_general(codes, context, (((1,), (0,)), ((), ())),
                              preferred_element_type=jnp.float32)
    out_ref[0, 0] = (out + bias * v) * d_inv


@jax.jit
def kernel(qk, v, anchors, W):
    b, h, n, c = qk.shape
    m = anchors.shape[2]
    nbits = W.shape[2]
    grid = (b, h)
    return pl.pallas_call(
        functools.partial(_head_kernel, n=n, nbits=nbits),
        grid=grid,
        in_specs=[
            pl.BlockSpec((1, 1, n, c), lambda i, j: (i, j, 0, 0)),
            pl.BlockSpec((1, 1, n, c), lambda i, j: (i, j, 0, 0)),
            pl.BlockSpec((1, 1, m, c), lambda i, j: (0, j, 0, 0)),
            pl.BlockSpec((1, m, nbits), lambda i, j: (j, 0, 0)),
        ],
        out_specs=pl.BlockSpec((1, 1, n, c), lambda i, j: (i, j, 0, 0)),
        out_shape=jax.ShapeDtypeStruct((b, h, n, c), jnp.float32),
        compiler_params=pltpu.CompilerParams(
            dimension_semantics=("parallel", "parallel"),
        ),
    )(qk, v, anchors, W)
